# baseline (speedup 1.0000x reference)
import functools
import math

import jax
import jax.numpy as jnp
from jax import lax
from jax.experimental import pallas as pl
from jax.experimental.pallas import tpu as pltpu
from jax.experimental.pallas import tpu_sc as plsc

SMOOTH = 0.1
CONF = 1.0 - SMOOTH

NC = 2
NS = 16
NW = NC * NS
LANES = 16

SC_ROWS = 50688
CR = 16
RING = 4

BLK_ROWS = 184
NSTREAMS = 4


def _sc_reduce_and_gather(xt, target, s_coef, g_coef):
    n = target.shape[0]
    b_per_w = n // NW
    rpw = SC_ROWS // NW
    nchunks = rpw // CR
    assert nchunks % RING == RING - 1
    mesh = plsc.VectorSubcoreMesh(core_axis_name="c", subcore_axis_name="s")

    @functools.partial(
        pl.kernel,
        mesh=mesh,
        out_type=jax.ShapeDtypeStruct((NW, LANES), jnp.float32),
        scratch_types=[
            pltpu.VMEM((b_per_w,), jnp.int32),
            pltpu.VMEM((b_per_w, n), jnp.float32),
            pltpu.VMEM((RING, CR, n), jnp.float32),
            pltpu.VMEM((LANES,), jnp.float32),
            pltpu.SemaphoreType.DMA,
            pltpu.SemaphoreType.DMA,
        ],
    )
    def sc_kernel(x_hbm, tgt_hbm, out_hbm, tgt_v, rows_v, buf_v, acc_v,
                  gsem, dsem):
        cid = lax.axis_index("c")
        sid = lax.axis_index("s")
        wid = sid * NC + cid
        lane_iota = lax.iota(jnp.int32, LANES)
        zeros16 = jnp.zeros((LANES,), jnp.float32)

        gbase = wid * b_per_w
        pltpu.sync_copy(tgt_hbm.at[pl.ds(gbase, b_per_w)], tgt_v)
        grows = pltpu.async_copy(x_hbm.at[tgt_v], rows_v, gsem)

        drow0 = wid * rpw

        def chunk_src(k):
            row = pl.multiple_of(drow0 + k * CR, 8)
            return x_hbm.at[pl.ds(row, CR), :]

        def buf_wait(b):
            pltpu.make_async_copy(chunk_src(0), buf_v.at[b], dsem).wait()

        def accum_buf(b, acc):
            def row_body(i, a):
                for cidx in range(n // LANES):
                    a = a + buf_v[b, i, pl.ds(cidx * LANES, LANES)]
                return a
            return lax.fori_loop(0, CR, row_body, acc)

        for p in range(RING - 1):
            pltpu.async_copy(chunk_src(p), buf_v.at[p], dsem)

        main_iters = (nchunks - (RING - 1)) // RING

        def ring_body(k2, acc):
            for b in range(RING):
                k = RING * k2 + b
                buf_wait(b)
                pltpu.async_copy(chunk_src(k + RING - 1),
                                 buf_v.at[(b + RING - 1) % RING], dsem)
                acc = accum_buf(b, acc)
            return acc

        dense = lax.fori_loop(0, main_iters, ring_body, zeros16)
        for k in range(RING * main_iters, nchunks):
            buf_wait(k % RING)
            dense = accum_buf(k % RING, dense)

        grows.wait()
        gacc = zeros16
        for e in range(b_per_w):
            col0 = gbase + (e & ~(LANES - 1))
            seg = rows_v[e, pl.ds(col0, LANES)]
            gacc = gacc + jnp.where(lane_iota == (e & (LANES - 1)), seg,
                                    0.0)
        acc_v[...] = dense * s_coef + gacc * g_coef
        pltpu.sync_copy(acc_v, out_hbm.at[wid])

    return sc_kernel(xt, target)


def kernel(x, target):
    n, size = x.shape
    s = SMOOTH / (size - 1)
    c0 = float(n * ((size - 1) * s * math.log(s) + CONF * math.log(CONF)))
    s_coef = float(s)
    g_coef = float(CONF - s)

    xt = x.T
    sc_part = _sc_reduce_and_gather(xt, target.astype(jnp.int32), s_coef,
                                    g_coef)

    grid = (size - SC_ROWS) // (BLK_ROWS * NSTREAMS)
    row_base = SC_ROWS // BLK_ROWS

    def tc_body(*refs):
        x_refs = refs[:NSTREAMS]
        o_ref = refs[NSTREAMS]
        i = pl.program_id(0)

        @pl.when(i == 0)
        def _init():
            o_ref[0, 0] = jnp.float32(0.0)

        part = x_refs[0][...]
        for r in x_refs[1:]:
            part = part + r[...]
        o_ref[0, 0] += jnp.sum(part)

        @pl.when(i == grid - 1)
        def _final():
            o_ref[0, 0] = c0 - s_coef * o_ref[0, 0]

    tc_sum = pl.pallas_call(
        tc_body,
        grid=(grid,),
        in_specs=[
            pl.BlockSpec(
                (BLK_ROWS, n),
                functools.partial(
                    lambda k, i: (row_base + NSTREAMS * i + k, 0), k))
            for k in range(NSTREAMS)
        ],
        out_specs=pl.BlockSpec((1, 1), lambda i: (0, 0),
                               memory_space=pltpu.SMEM),
        out_shape=jax.ShapeDtypeStruct((1, 1), jnp.float32),
    )(*([xt] * NSTREAMS))

    def combine_body(t_ref, g_ref, o_ref):
        o_ref[0, 0] = t_ref[0, 0] - jnp.sum(g_ref[...])

    out = pl.pallas_call(
        combine_body,
        in_specs=[
            pl.BlockSpec(memory_space=pltpu.SMEM),
            pl.BlockSpec(memory_space=pltpu.VMEM),
        ],
        out_specs=pl.BlockSpec(memory_space=pltpu.SMEM),
        out_shape=jax.ShapeDtypeStruct((1, 1), jnp.float32),
    )(tc_sum, sc_part)
    return out.reshape(())


# speedup vs baseline: 1.0467x; 1.0467x over previous
import functools
import math

import jax
import jax.numpy as jnp
from jax import lax
from jax.experimental import pallas as pl
from jax.experimental.pallas import tpu as pltpu
from jax.experimental.pallas import tpu_sc as plsc

SMOOTH = 0.1
CONF = 1.0 - SMOOTH

NC = 2
NS = 16
NW = NC * NS
LANES = 16

SC_ROWS = 48640
CR = 16
RING = 4

BLK_ROWS = 120
NSTREAMS = 4


def _sc_reduce_and_gather(xt, target, s_coef, g_coef):
    n = target.shape[0]
    tc_rows = xt.shape[0] - SC_ROWS
    b_per_w = n // NW
    rpw = SC_ROWS // NW
    nchunks = rpw // CR
    assert nchunks % RING == RING - 1
    mesh = plsc.VectorSubcoreMesh(core_axis_name="c", subcore_axis_name="s")

    @functools.partial(
        pl.kernel,
        mesh=mesh,
        out_type=jax.ShapeDtypeStruct((NW, LANES), jnp.float32),
        scratch_types=[
            pltpu.VMEM((b_per_w,), jnp.int32),
            pltpu.VMEM((b_per_w, n), jnp.float32),
            pltpu.VMEM((RING, CR, n), jnp.float32),
            pltpu.VMEM((LANES,), jnp.float32),
            pltpu.SemaphoreType.DMA,
            pltpu.SemaphoreType.DMA,
        ],
    )
    def sc_kernel(x_hbm, tgt_hbm, out_hbm, tgt_v, rows_v, buf_v, acc_v,
                  gsem, dsem):
        cid = lax.axis_index("c")
        sid = lax.axis_index("s")
        wid = sid * NC + cid
        lane_iota = lax.iota(jnp.int32, LANES)
        zeros16 = jnp.zeros((LANES,), jnp.float32)

        gbase = wid * b_per_w
        pltpu.sync_copy(tgt_hbm.at[pl.ds(gbase, b_per_w)], tgt_v)
        grows = pltpu.async_copy(x_hbm.at[tgt_v], rows_v, gsem)

        drow0 = tc_rows + wid * rpw

        def chunk_src(k):
            row = pl.multiple_of(drow0 + k * CR, 8)
            return x_hbm.at[pl.ds(row, CR), :]

        def buf_wait(b):
            pltpu.make_async_copy(chunk_src(0), buf_v.at[b], dsem).wait()

        def accum_buf(b, acc):
            def row_body(i, a):
                for cidx in range(n // LANES):
                    a = a + buf_v[b, i, pl.ds(cidx * LANES, LANES)]
                return a
            return lax.fori_loop(0, CR, row_body, acc)

        for p in range(RING - 1):
            pltpu.async_copy(chunk_src(p), buf_v.at[p], dsem)

        main_iters = (nchunks - (RING - 1)) // RING

        def ring_body(k2, acc):
            for b in range(RING):
                k = RING * k2 + b
                buf_wait(b)
                pltpu.async_copy(chunk_src(k + RING - 1),
                                 buf_v.at[(b + RING - 1) % RING], dsem)
                acc = accum_buf(b, acc)
            return acc

        dense = lax.fori_loop(0, main_iters, ring_body, zeros16)
        for k in range(RING * main_iters, nchunks):
            buf_wait(k % RING)
            dense = accum_buf(k % RING, dense)

        grows.wait()
        gacc = zeros16
        for e in range(b_per_w):
            col0 = gbase + (e & ~(LANES - 1))
            seg = rows_v[e, pl.ds(col0, LANES)]
            gacc = gacc + jnp.where(lane_iota == (e & (LANES - 1)), seg,
                                    0.0)
        acc_v[...] = dense * s_coef + gacc * g_coef
        pltpu.sync_copy(acc_v, out_hbm.at[wid])

    return sc_kernel(xt, target)


def kernel(x, target):
    n, size = x.shape
    s = SMOOTH / (size - 1)
    c0 = float(n * ((size - 1) * s * math.log(s) + CONF * math.log(CONF)))
    s_coef = float(s)
    g_coef = float(CONF - s)

    xt = x.T
    sc_part = _sc_reduce_and_gather(xt, target.astype(jnp.int32), s_coef,
                                    g_coef)

    grid = (size - SC_ROWS) // (BLK_ROWS * NSTREAMS)

    def tc_body(*refs):
        x_refs = refs[:NSTREAMS]
        o_ref = refs[NSTREAMS]
        i = pl.program_id(0)

        @pl.when(i == 0)
        def _init():
            o_ref[0, 0] = jnp.float32(0.0)

        part = x_refs[0][...]
        for r in x_refs[1:]:
            part = part + r[...]
        o_ref[0, 0] += jnp.sum(part)

        @pl.when(i == grid - 1)
        def _final():
            o_ref[0, 0] = c0 - s_coef * o_ref[0, 0]

    tc_sum = pl.pallas_call(
        tc_body,
        grid=(grid,),
        in_specs=[
            pl.BlockSpec(
                (BLK_ROWS, n),
                functools.partial(
                    lambda k, i: (NSTREAMS * i + k, 0), k))
            for k in range(NSTREAMS)
        ],
        out_specs=pl.BlockSpec((1, 1), lambda i: (0, 0),
                               memory_space=pltpu.SMEM),
        out_shape=jax.ShapeDtypeStruct((1, 1), jnp.float32),
    )(*([xt] * NSTREAMS))

    def combine_body(t_ref, g_ref, o_ref):
        o_ref[0, 0] = t_ref[0, 0] - jnp.sum(g_ref[...])

    out = pl.pallas_call(
        combine_body,
        in_specs=[
            pl.BlockSpec(memory_space=pltpu.SMEM),
            pl.BlockSpec(memory_space=pltpu.VMEM),
        ],
        out_specs=pl.BlockSpec(memory_space=pltpu.SMEM),
        out_shape=jax.ShapeDtypeStruct((1, 1), jnp.float32),
    )(tc_sum, sc_part)
    return out.reshape(())


# speedup vs baseline: 1.0532x; 1.0062x over previous
import functools
import math

import jax
import jax.numpy as jnp
from jax import lax
from jax.experimental import pallas as pl
from jax.experimental.pallas import tpu as pltpu
from jax.experimental.pallas import tpu_sc as plsc

SMOOTH = 0.1
CONF = 1.0 - SMOOTH

NC = 2
NS = 16
NW = NC * NS
LANES = 16

SC_ROWS = 48640
CR = 16
RING = 4

BLK_ROWS = 80
NSTREAMS = 6


def _sc_reduce_and_gather(xt, target, s_coef, g_coef):
    n = target.shape[0]
    tc_rows = xt.shape[0] - SC_ROWS
    b_per_w = n // NW
    rpw = SC_ROWS // NW
    nchunks = rpw // CR
    assert nchunks % RING == RING - 1
    mesh = plsc.VectorSubcoreMesh(core_axis_name="c", subcore_axis_name="s")

    @functools.partial(
        pl.kernel,
        mesh=mesh,
        out_type=jax.ShapeDtypeStruct((NW, LANES), jnp.float32),
        scratch_types=[
            pltpu.VMEM((b_per_w,), jnp.int32),
            pltpu.VMEM((b_per_w, n), jnp.float32),
            pltpu.VMEM((RING, CR, n), jnp.float32),
            pltpu.VMEM((LANES,), jnp.float32),
            pltpu.SemaphoreType.DMA,
            pltpu.SemaphoreType.DMA,
        ],
    )
    def sc_kernel(x_hbm, tgt_hbm, out_hbm, tgt_v, rows_v, buf_v, acc_v,
                  gsem, dsem):
        cid = lax.axis_index("c")
        sid = lax.axis_index("s")
        wid = sid * NC + cid
        lane_iota = lax.iota(jnp.int32, LANES)
        zeros16 = jnp.zeros((LANES,), jnp.float32)

        gbase = wid * b_per_w
        pltpu.sync_copy(tgt_hbm.at[pl.ds(gbase, b_per_w)], tgt_v)
        grows = pltpu.async_copy(x_hbm.at[tgt_v], rows_v, gsem)

        drow0 = tc_rows + wid * rpw

        def chunk_src(k):
            row = pl.multiple_of(drow0 + k * CR, 8)
            return x_hbm.at[pl.ds(row, CR), :]

        def buf_wait(b):
            pltpu.make_async_copy(chunk_src(0), buf_v.at[b], dsem).wait()

        def accum_buf(b, acc):
            def row_body(i, a):
                for cidx in range(n // LANES):
                    a = a + buf_v[b, i, pl.ds(cidx * LANES, LANES)]
                return a
            return lax.fori_loop(0, CR, row_body, acc)

        for p in range(RING - 1):
            pltpu.async_copy(chunk_src(p), buf_v.at[p], dsem)

        main_iters = (nchunks - (RING - 1)) // RING

        def ring_body(k2, acc):
            for b in range(RING):
                k = RING * k2 + b
                buf_wait(b)
                pltpu.async_copy(chunk_src(k + RING - 1),
                                 buf_v.at[(b + RING - 1) % RING], dsem)
                acc = accum_buf(b, acc)
            return acc

        dense = lax.fori_loop(0, main_iters, ring_body, zeros16)
        for k in range(RING * main_iters, nchunks):
            buf_wait(k % RING)
            dense = accum_buf(k % RING, dense)

        grows.wait()
        gacc = zeros16
        for e in range(b_per_w):
            col0 = gbase + (e & ~(LANES - 1))
            seg = rows_v[e, pl.ds(col0, LANES)]
            gacc = gacc + jnp.where(lane_iota == (e & (LANES - 1)), seg,
                                    0.0)
        acc_v[...] = dense * s_coef + gacc * g_coef
        pltpu.sync_copy(acc_v, out_hbm.at[wid])

    return sc_kernel(xt, target)


def kernel(x, target):
    n, size = x.shape
    s = SMOOTH / (size - 1)
    c0 = float(n * ((size - 1) * s * math.log(s) + CONF * math.log(CONF)))
    s_coef = float(s)
    g_coef = float(CONF - s)

    xt = x.T
    sc_part = _sc_reduce_and_gather(xt, target.astype(jnp.int32), s_coef,
                                    g_coef)

    grid = (size - SC_ROWS) // (BLK_ROWS * NSTREAMS)

    def tc_body(*refs):
        x_refs = refs[:NSTREAMS]
        o_ref = refs[NSTREAMS]
        i = pl.program_id(0)

        @pl.when(i == 0)
        def _init():
            o_ref[0, 0] = jnp.float32(0.0)

        part = x_refs[0][...]
        for r in x_refs[1:]:
            part = part + r[...]
        o_ref[0, 0] += jnp.sum(part)

        @pl.when(i == grid - 1)
        def _final():
            o_ref[0, 0] = c0 - s_coef * o_ref[0, 0]

    tc_sum = pl.pallas_call(
        tc_body,
        grid=(grid,),
        in_specs=[
            pl.BlockSpec(
                (BLK_ROWS, n),
                functools.partial(
                    lambda k, i: (NSTREAMS * i + k, 0), k))
            for k in range(NSTREAMS)
        ],
        out_specs=pl.BlockSpec((1, 1), lambda i: (0, 0),
                               memory_space=pltpu.SMEM),
        out_shape=jax.ShapeDtypeStruct((1, 1), jnp.float32),
    )(*([xt] * NSTREAMS))

    def combine_body(t_ref, g_ref, o_ref):
        o_ref[0, 0] = t_ref[0, 0] - jnp.sum(g_ref[...])

    out = pl.pallas_call(
        combine_body,
        in_specs=[
            pl.BlockSpec(memory_space=pltpu.SMEM),
            pl.BlockSpec(memory_space=pltpu.VMEM),
        ],
        out_specs=pl.BlockSpec(memory_space=pltpu.SMEM),
        out_shape=jax.ShapeDtypeStruct((1, 1), jnp.float32),
    )(tc_sum, sc_part)
    return out.reshape(())
